# baseline (speedup 1.0000x reference)
import functools

import jax
import jax.numpy as jnp
from jax.experimental import pallas as pl

B, M, K, D = 8192, 4, 1024, 64
BLK = 512


def _stage_body(first, x_ref, xpm_ref, att_ref, ct_ref, idx_ref, xpm_out_ref,
                xr_ref, rxn_ref, osum_ref, ssum_ref, msum_ref):
    i = pl.program_id(0)
    ct = ct_ref[...]
    att = att_ref[...]
    if first:
        rx = x_ref[...]
        xpm_prev = None
    else:
        xpm_prev = xpm_ref[...]
        rx = x_ref[...] - xpm_prev

    m = jnp.max(att, axis=1, keepdims=True)
    e = jnp.exp(att - m)
    s = jnp.sum(e, axis=1, keepdims=True)
    soft = e / s

    smax = jnp.max(soft, axis=1, keepdims=True)
    iota = jax.lax.broadcasted_iota(jnp.int32, soft.shape, 1)
    idx = jnp.min(jnp.where(soft == smax, iota, K), axis=1,
                  keepdims=True)
    idx_ref[...] = idx

    cbt = ct.astype(jnp.bfloat16)
    oh = (iota == idx).astype(jnp.bfloat16)
    r1col = (jnp.float32(1.0) - smax) + smax
    xp = jax.lax.dot_general(
        oh, cbt, (((1,), (1,)), ((), ())),
        preferred_element_type=jnp.float32) * r1col

    softc = jax.lax.dot_general(
        soft.astype(jnp.bfloat16), cbt, (((1,), (1,)), ((), ())),
        preferred_element_type=jnp.float32)

    sm = smax[:, 0]
    r1 = r1col[:, 0]
    match_part = jnp.sum(jnp.sum(soft * soft, axis=1) - sm * sm
                         + (r1 - sm) * (r1 - sm))
    sout_part = jnp.sum((softc - rx) ** 2)
    out_part = jnp.sum((xp - rx) ** 2)

    xr_ref[...] = xp
    xpm = xp if first else xpm_prev + xp
    xpm_out_ref[...] = xpm
    rxn_ref[...] = x_ref[...] - xpm

    @pl.when(i == 0)
    def _init():
        osum_ref[...] = out_part[None, None]
        ssum_ref[...] = sout_part[None, None]
        msum_ref[...] = match_part[None, None]

    @pl.when(i != 0)
    def _acc():
        osum_ref[...] += out_part[None, None]
        ssum_ref[...] += sout_part[None, None]
        msum_ref[...] += match_part[None, None]


def _stage_call(first, x, xpm, att, ct):
    grid = (B // BLK,)
    row_spec = pl.BlockSpec((BLK, D), lambda i: (i, 0))
    att_spec = pl.BlockSpec((BLK, K), lambda i: (i, 0))
    scal_spec = pl.BlockSpec((1, 1), lambda i: (0, 0))
    in_specs = [row_spec, row_spec, att_spec,
                pl.BlockSpec((D, K), lambda i: (0, 0))]
    args = [x, x if first else xpm, att, ct]
    out_shape = [
        jax.ShapeDtypeStruct((B, 1), jnp.int32),
        jax.ShapeDtypeStruct((B, D), jnp.float32),
        jax.ShapeDtypeStruct((B, D), jnp.float32),
        jax.ShapeDtypeStruct((B, D), jnp.float32),
        jax.ShapeDtypeStruct((1, 1), jnp.float32),
        jax.ShapeDtypeStruct((1, 1), jnp.float32),
        jax.ShapeDtypeStruct((1, 1), jnp.float32),
    ]
    out_specs = [
        pl.BlockSpec((BLK, 1), lambda i: (i, 0)),
        row_spec, row_spec, row_spec,
        scal_spec, scal_spec, scal_spec,
    ]
    return pl.pallas_call(
        functools.partial(_stage_body, first),
        grid=grid, in_specs=in_specs, out_specs=out_specs,
        out_shape=out_shape)(*args)


def _combine_body(os_ref, ss_ref, ms_ref, w_ref, out_ref):
    scale = jnp.float32(1.0 / (B * D))
    mscale = jnp.float32(1.0 / (B * K))
    w = w_ref[0, :]
    lquanH = jnp.sum(w * (os_ref[0, :] * scale))
    lquan = jnp.sum(w * (ss_ref[0, :] * scale))
    lmatch = jnp.sum(w * (ms_ref[0, :] * mscale))
    out_ref[...] = (lquanH + 0.1 * lmatch + lquan)[None, None]


def kernel(x, center, weight):
    idxs, xrs, rxs = [], [], []
    osums, ssums, msums = [], [], []
    xpm = None
    rx = x
    for j in range(M):
        att = -jnp.sqrt(jnp.sum(jnp.square(rx[:, None, :] - center[j]),
                                axis=-1))
        rxs.append(rx)
        idx, xpm, xr, rx, osum, ssum, msum = _stage_call(
            j == 0, x, xpm, att, center[j].T)
        idxs.append(idx)
        xrs.append(xr)
        osums.append(osum)
        ssums.append(ssum)
        msums.append(msum)

    os_ = jnp.concatenate(osums, axis=1)
    ss_ = jnp.concatenate(ssums, axis=1)
    ms_ = jnp.concatenate(msums, axis=1)
    out = pl.pallas_call(
        _combine_body,
        out_shape=jax.ShapeDtypeStruct((1, 1), jnp.float32),
    )(os_, ss_, ms_, weight)

    X_r_matrix = jnp.stack(xrs, axis=1)
    X_p_matrix = jnp.stack(rxs, axis=1)
    codes = jnp.concatenate(idxs, axis=1)
    codebooks = center.reshape(M * K, D)
    return (X_r_matrix, X_p_matrix, xpm, xpm, codebooks, codes, out)


# speedup vs baseline: 1.0073x; 1.0073x over previous
import functools

import jax
import jax.numpy as jnp
from jax.experimental import pallas as pl

B, M, K, D = 8192, 4, 1024, 64
BLK = 1024


def _stage_body(first, x_ref, xpm_ref, att_ref, ct_ref, idx_ref, xpm_out_ref,
                xr_ref, rxn_ref, osum_ref, ssum_ref, msum_ref):
    i = pl.program_id(0)
    ct = ct_ref[...]
    att = att_ref[...]
    if first:
        rx = x_ref[...]
        xpm_prev = None
    else:
        xpm_prev = xpm_ref[...]
        rx = x_ref[...] - xpm_prev

    m = jnp.max(att, axis=1, keepdims=True)
    e = jnp.exp(att - m)
    s = jnp.sum(e, axis=1, keepdims=True)
    soft = e / s

    smax = jnp.max(soft, axis=1, keepdims=True)
    iota = jax.lax.broadcasted_iota(jnp.int32, soft.shape, 1)
    idx = jnp.min(jnp.where(soft == smax, iota, K), axis=1,
                  keepdims=True)
    idx_ref[...] = idx

    cbt = ct.astype(jnp.bfloat16)
    oh = (iota == idx).astype(jnp.bfloat16)
    r1col = (jnp.float32(1.0) - smax) + smax
    xp = jax.lax.dot_general(
        oh, cbt, (((1,), (1,)), ((), ())),
        preferred_element_type=jnp.float32) * r1col

    softc = jax.lax.dot_general(
        soft.astype(jnp.bfloat16), cbt, (((1,), (1,)), ((), ())),
        preferred_element_type=jnp.float32)

    sm = smax[:, 0]
    r1 = r1col[:, 0]
    match_part = jnp.sum(jnp.sum(soft * soft, axis=1) - sm * sm
                         + (r1 - sm) * (r1 - sm))
    sout_part = jnp.sum((softc - rx) ** 2)
    out_part = jnp.sum((xp - rx) ** 2)

    xr_ref[...] = xp
    xpm = xp if first else xpm_prev + xp
    xpm_out_ref[...] = xpm
    rxn_ref[...] = x_ref[...] - xpm

    @pl.when(i == 0)
    def _init():
        osum_ref[...] = out_part[None, None]
        ssum_ref[...] = sout_part[None, None]
        msum_ref[...] = match_part[None, None]

    @pl.when(i != 0)
    def _acc():
        osum_ref[...] += out_part[None, None]
        ssum_ref[...] += sout_part[None, None]
        msum_ref[...] += match_part[None, None]


def _stage_call(first, x, xpm, att, ct):
    grid = (B // BLK,)
    row_spec = pl.BlockSpec((BLK, D), lambda i: (i, 0))
    att_spec = pl.BlockSpec((BLK, K), lambda i: (i, 0))
    scal_spec = pl.BlockSpec((1, 1), lambda i: (0, 0))
    in_specs = [row_spec, row_spec, att_spec,
                pl.BlockSpec((D, K), lambda i: (0, 0))]
    args = [x, x if first else xpm, att, ct]
    out_shape = [
        jax.ShapeDtypeStruct((B, 1), jnp.int32),
        jax.ShapeDtypeStruct((B, D), jnp.float32),
        jax.ShapeDtypeStruct((B, D), jnp.float32),
        jax.ShapeDtypeStruct((B, D), jnp.float32),
        jax.ShapeDtypeStruct((1, 1), jnp.float32),
        jax.ShapeDtypeStruct((1, 1), jnp.float32),
        jax.ShapeDtypeStruct((1, 1), jnp.float32),
    ]
    out_specs = [
        pl.BlockSpec((BLK, 1), lambda i: (i, 0)),
        row_spec, row_spec, row_spec,
        scal_spec, scal_spec, scal_spec,
    ]
    return pl.pallas_call(
        functools.partial(_stage_body, first),
        grid=grid, in_specs=in_specs, out_specs=out_specs,
        out_shape=out_shape)(*args)


def _combine_body(os_ref, ss_ref, ms_ref, w_ref, out_ref):
    scale = jnp.float32(1.0 / (B * D))
    mscale = jnp.float32(1.0 / (B * K))
    w = w_ref[0, :]
    lquanH = jnp.sum(w * (os_ref[0, :] * scale))
    lquan = jnp.sum(w * (ss_ref[0, :] * scale))
    lmatch = jnp.sum(w * (ms_ref[0, :] * mscale))
    out_ref[...] = (lquanH + 0.1 * lmatch + lquan)[None, None]


def kernel(x, center, weight):
    idxs, xrs, rxs = [], [], []
    osums, ssums, msums = [], [], []
    xpm = None
    rx = x
    for j in range(M):
        att = -jnp.sqrt(jnp.sum(jnp.square(rx[:, None, :] - center[j]),
                                axis=-1))
        rxs.append(rx)
        idx, xpm, xr, rx, osum, ssum, msum = _stage_call(
            j == 0, x, xpm, att, center[j].T)
        idxs.append(idx)
        xrs.append(xr)
        osums.append(osum)
        ssums.append(ssum)
        msums.append(msum)

    os_ = jnp.concatenate(osums, axis=1)
    ss_ = jnp.concatenate(ssums, axis=1)
    ms_ = jnp.concatenate(msums, axis=1)
    out = pl.pallas_call(
        _combine_body,
        out_shape=jax.ShapeDtypeStruct((1, 1), jnp.float32),
    )(os_, ss_, ms_, weight)

    X_r_matrix = jnp.stack(xrs, axis=1)
    X_p_matrix = jnp.stack(rxs, axis=1)
    codes = jnp.concatenate(idxs, axis=1)
    codebooks = center.reshape(M * K, D)
    return (X_r_matrix, X_p_matrix, xpm, xpm, codebooks, codes, out)
